# Initial kernel scaffold; baseline (speedup 1.0000x reference)
import jax
import jax.numpy as jnp
from jax.experimental import pallas as pl


def kernel(x, W_sel, W1, b1, W2, b2):
    raise NotImplementedError("write your pallas kernel here")


# speedup vs baseline: 1.2786x; 1.2786x over previous
import jax
import jax.numpy as jnp
from jax.experimental import pallas as pl

_T = 256


def _moe_dense_kernel(x_ref, wsel_ref, w1_ref, b1_ref, w2_ref, b2_ref, o_ref):
    E = wsel_ref.shape[0]
    xt = x_ref[...]
    xb = xt.astype(jnp.bfloat16)
    scores = jax.lax.dot_general(
        xb, wsel_ref[...].astype(jnp.bfloat16), (((1,), (1,)), ((), ())),
        preferred_element_type=jnp.float32)
    probs = jax.nn.sigmoid(scores)
    maxv = jnp.max(probs, axis=1, keepdims=True)
    is_max = probs == maxv
    lane = jax.lax.broadcasted_iota(jnp.int32, probs.shape, 1)
    first_lane = jnp.min(jnp.where(is_max, lane, probs.shape[1]), axis=1, keepdims=True)
    first = lane == first_lane
    gates = jnp.where(first, maxv, 0.0)
    acc = jnp.zeros(o_ref.shape, jnp.float32)
    for e in range(E):
        h = jax.lax.dot_general(
            xb, w1_ref[e].astype(jnp.bfloat16), (((1,), (0,)), ((), ())),
            preferred_element_type=jnp.float32)
        h = jnp.maximum(h + b1_ref[e][None, :], 0.0)
        y = jax.lax.dot_general(
            h.astype(jnp.bfloat16), w2_ref[e].astype(jnp.bfloat16),
            (((1,), (0,)), ((), ())),
            preferred_element_type=jnp.float32) + b2_ref[e][None, :]
        acc = acc + gates[:, e:e + 1] * y
    o_ref[...] = acc


def kernel(x, W_sel, W1, b1, W2, b2):
    B, S, D = x.shape
    E, _, H = W1.shape
    xf = x.reshape(S, D)
    out = pl.pallas_call(
        _moe_dense_kernel,
        grid=(S // _T,),
        in_specs=[
            pl.BlockSpec((_T, D), lambda i: (i, 0)),
            pl.BlockSpec((E, D), lambda i: (0, 0)),
            pl.BlockSpec((E, D, H), lambda i: (0, 0, 0)),
            pl.BlockSpec((E, H), lambda i: (0, 0)),
            pl.BlockSpec((E, H, D), lambda i: (0, 0, 0)),
            pl.BlockSpec((E, D), lambda i: (0, 0)),
        ],
        out_specs=pl.BlockSpec((_T, D), lambda i: (i, 0)),
        out_shape=jax.ShapeDtypeStruct((S, D), jnp.float32),
    )(xf, W_sel, W1, b1, W2, b2)
    return out.reshape(B, S, D)
